# Initial kernel scaffold; baseline (speedup 1.0000x reference)
import jax
import jax.numpy as jnp
from jax.experimental import pallas as pl


def kernel(x):
    raise NotImplementedError("write your pallas kernel here")


# speedup vs baseline: 1.8908x; 1.8908x over previous
import functools

import jax
import jax.numpy as jnp
from jax import lax
from jax.experimental import pallas as pl
from jax.experimental.pallas import tpu as pltpu
from jax.experimental.pallas import tpu_sc as plsc

_K = 512
_NITERS = 10
_D = 128

_BN = 1024

_NC = 2
_NS = 16
_NW = _NC * _NS
_CH = 128


def _assign_body(x_ref, c_ref, out_ref, cnt_ref):
    xb = x_ref[...]
    c = c_ref[...]
    c2 = jnp.sum(c * c, axis=1)
    x2 = jnp.sum(xb * xb, axis=1, keepdims=True)
    prod = lax.dot_general(xb, c, (((1,), (1,)), ((), ())),
                           preferred_element_type=jnp.float32)
    d = x2 - 2.0 * prod + c2[None, :]
    m = jnp.min(d, axis=1, keepdims=True)
    ids = lax.broadcasted_iota(jnp.int32, d.shape, 1)
    chosen = jnp.min(jnp.where(d == m, ids, _K), axis=1)
    out_ref[...] = chosen.astype(jnp.int32)

    onehot = (ids == chosen[:, None]).astype(jnp.float32)
    blk_counts = jnp.sum(onehot, axis=0)

    @pl.when(pl.program_id(0) == 0)
    def _():
        cnt_ref[...] = jnp.zeros_like(cnt_ref)

    cnt_ref[...] += blk_counts


def _assign(x, centroids):
    n, d = x.shape
    nb = n // _BN
    return pl.pallas_call(
        _assign_body,
        grid=(nb,),
        in_specs=[
            pl.BlockSpec((_BN, d), lambda i: (i, 0)),
            pl.BlockSpec((_K, d), lambda i: (0, 0)),
        ],
        out_specs=[
            pl.BlockSpec((_BN,), lambda i: (i,)),
            pl.BlockSpec((_K,), lambda i: (0,)),
        ],
        out_shape=[
            jax.ShapeDtypeStruct((n,), jnp.int32),
            jax.ShapeDtypeStruct((_K,), jnp.float32),
        ],
    )(x, centroids)


def _sc_update_body(x, clus, zeros, out, buf, idx, acc):
    cid = lax.axis_index("c")
    sid = lax.axis_index("s")
    wid = sid * _NC + cid
    rows_per_w = 65536 // _NW
    base = wid * rows_per_w
    rows_per_tile = _K // _NS

    pltpu.sync_copy(zeros.at[pl.ds(sid * rows_per_tile, rows_per_tile)],
                    acc.at[pl.ds(sid * rows_per_tile, rows_per_tile)])
    plsc.subcore_barrier()

    for g in range(rows_per_w // _CH):
        off = base + g * _CH
        pltpu.sync_copy(clus.at[pl.ds(off, _CH)], idx)
        pltpu.sync_copy(x.at[pl.ds(off, _CH)], buf)
        pltpu.sync_copy(buf, acc.at[idx], add=True)
    plsc.subcore_barrier()

    pltpu.sync_copy(acc.at[pl.ds(sid * rows_per_tile, rows_per_tile)],
                    out.at[cid, pl.ds(sid * rows_per_tile, rows_per_tile)])


@functools.cache
def _make_sc_update():
    return pl.kernel(
        _sc_update_body,
        out_type=jax.ShapeDtypeStruct((_NC, _K, _D), jnp.float32),
        mesh=plsc.VectorSubcoreMesh(core_axis_name="c", subcore_axis_name="s"),
        scratch_types=[
            pltpu.VMEM((_CH, _D), jnp.float32),
            pltpu.VMEM((_CH,), jnp.int32),
            pltpu.VMEM_SHARED((_K, _D), jnp.float32),
        ],
    )


def kernel(x):
    n, d = x.shape
    zeros = jnp.zeros((_K, _D), jnp.float32)
    c0 = x[:_K, :]

    def body(_, carry):
        centroids, _, _ = carry
        clusters, counts = _assign(x, centroids)
        part = _make_sc_update()(x, clusters, zeros)
        sums = part[0] + part[1]
        centroids = sums / counts[:, None]
        return centroids, clusters, counts

    init = (c0, jnp.zeros((n,), jnp.int32), jnp.zeros((_K,), jnp.float32))
    centroids, clusters, counts = lax.fori_loop(0, _NITERS, body, init)
    return clusters.astype(jnp.int64), centroids, counts
